# baseline (speedup 1.0000x reference)
import jax
import jax.numpy as jnp
from jax.experimental import pallas as pl
from jax.experimental.pallas import tpu as pltpu

_EMBEDDING_DIM = 1024
_N_CHUNKS = 8


def _dma_body(t_ref, o_ref, sems):
    rows = t_ref.shape[0]
    chunk = rows // _N_CHUNKS
    copies = []
    for c in range(_N_CHUNKS):
        sl = pl.ds(c * chunk, chunk)
        cp = pltpu.make_async_copy(t_ref.at[sl], o_ref.at[0].at[sl], sems.at[c])
        cp.start()
        copies.append(cp)
    for cp in copies:
        cp.wait()


def kernel(input, table):
    seq_len = input.shape[-1]
    out = pl.pallas_call(
        _dma_body,
        in_specs=[pl.BlockSpec(memory_space=pl.ANY)],
        out_specs=pl.BlockSpec(memory_space=pl.ANY),
        out_shape=jax.ShapeDtypeStruct((1, seq_len, _EMBEDDING_DIM), table.dtype),
        scratch_shapes=[pltpu.SemaphoreType.DMA((_N_CHUNKS,))],
    )(table)
    return out


# speedup vs baseline: 41.1370x; 41.1370x over previous
import jax
import jax.numpy as jnp
from jax.experimental import pallas as pl

_EMBEDDING_DIM = 1024
_BLOCK_ROWS = 512


def _copy_body(t_ref, o_ref):
    o_ref[0] = t_ref[...]


def kernel(input, table):
    seq_len = input.shape[-1]
    grid = (seq_len // _BLOCK_ROWS,)
    out = pl.pallas_call(
        _copy_body,
        grid=grid,
        in_specs=[
            pl.BlockSpec((_BLOCK_ROWS, _EMBEDDING_DIM), lambda i: (i, 0)),
        ],
        out_specs=pl.BlockSpec((1, _BLOCK_ROWS, _EMBEDDING_DIM), lambda i: (0, i, 0)),
        out_shape=jax.ShapeDtypeStruct((1, seq_len, _EMBEDDING_DIM), table.dtype),
    )(table)
    return out


# speedup vs baseline: 47.7060x; 1.1597x over previous
import jax
import jax.numpy as jnp
from jax.experimental import pallas as pl

_EMBEDDING_DIM = 1024
_BLOCK_ROWS = 2048


def _copy_body(t_ref, o_ref):
    o_ref[0] = t_ref[...]


def kernel(input, table):
    seq_len = input.shape[-1]
    grid = (seq_len // _BLOCK_ROWS,)
    out = pl.pallas_call(
        _copy_body,
        grid=grid,
        in_specs=[
            pl.BlockSpec((_BLOCK_ROWS, _EMBEDDING_DIM), lambda i: (i, 0)),
        ],
        out_specs=pl.BlockSpec((1, _BLOCK_ROWS, _EMBEDDING_DIM), lambda i: (0, i, 0)),
        out_shape=jax.ShapeDtypeStruct((1, seq_len, _EMBEDDING_DIM), table.dtype),
    )(table)
    return out


# speedup vs baseline: 47.9475x; 1.0051x over previous
import jax
import jax.numpy as jnp
from jax.experimental import pallas as pl
from jax.experimental.pallas import tpu as pltpu

_EMBEDDING_DIM = 1024
_BLOCK_ROWS = 2048


def _copy_body(t_ref, o_ref):
    o_ref[0] = t_ref[...]


def kernel(input, table):
    seq_len = input.shape[-1]
    grid = (seq_len // _BLOCK_ROWS,)
    out = pl.pallas_call(
        _copy_body,
        grid=grid,
        in_specs=[
            pl.BlockSpec((_BLOCK_ROWS, _EMBEDDING_DIM), lambda i: (i, 0)),
        ],
        out_specs=pl.BlockSpec((1, _BLOCK_ROWS, _EMBEDDING_DIM), lambda i: (0, i, 0)),
        out_shape=jax.ShapeDtypeStruct((1, seq_len, _EMBEDDING_DIM), table.dtype),
        compiler_params=pltpu.CompilerParams(
            dimension_semantics=("parallel",),
        ),
    )(table)
    return out
